# baseline (device time: 39295 ns/iter reference)
import jax
import jax.numpy as jnp
from jax import lax
from jax.experimental import pallas as pl
from jax.experimental.pallas import tpu as pltpu

N_DEV = 4
SQ = 256
HALF = 128
D_MODEL = 1024
H = 8
DH = 128
KSTAGE = 512
KSPAN = 384
SCALE = 0.08838834764831843
NEG = -1e9


def kernel(x, Wq, K_ext, V_ext, Wo):
    def body(x_hbm, wq_hbm, k_hbm, v_hbm, wo_hbm, out_ref,
             x_vmem, wq_vmem, wo_vmem, k_stage, v_stage,
             x_rel, partial, rs_buf,
             in_sems, k_sems, v_sems, ag_send, ag_recv, rs_send, rs_recv):
        my = lax.axis_index("i")

        def chunk_j(r):
            return lax.rem(my + N_DEV - r, N_DEV)

        def key_start(j):
            return jnp.maximum(0, j * SQ - HALF)

        xcopy = pltpu.make_async_copy(x_hbm.at[0], x_vmem, in_sems.at[0])
        wqcopy = pltpu.make_async_copy(wq_hbm, wq_vmem, in_sems.at[1])
        wocopy = pltpu.make_async_copy(wo_hbm, wo_vmem, in_sems.at[2])
        xcopy.start()
        wqcopy.start()

        kv = []
        for r in range(N_DEV):
            ks = key_start(chunk_j(r))
            kc = pltpu.make_async_copy(
                k_hbm.at[0, pl.ds(ks, KSTAGE), pl.ds(my * H, H), :],
                k_stage.at[r],
                k_sems.at[r],
            )
            vc = pltpu.make_async_copy(
                v_hbm.at[0, pl.ds(ks, KSTAGE), pl.ds(my * H, H), :],
                v_stage.at[r],
                v_sems.at[r],
            )
            kv.append((kc, vc))

        kv[0][0].start()
        kv[0][1].start()
        wocopy.start()
        for r in range(1, N_DEV):
            kv[r][0].start()
            kv[r][1].start()

        barrier = pltpu.get_barrier_semaphore()
        for d in range(1, N_DEV):
            pl.semaphore_signal(
                barrier, inc=1,
                device_id=((my + d) % N_DEV,),
                device_id_type=pl.DeviceIdType.MESH,
            )

        xcopy.wait()
        x_rel[0] = x_vmem[...].astype(jnp.bfloat16)

        pl.semaphore_wait(barrier, N_DEV - 1)

        ag = []
        for d in range(1, N_DEV):
            desc = pltpu.make_async_remote_copy(
                src_ref=x_rel.at[0],
                dst_ref=x_rel.at[d],
                send_sem=ag_send.at[d - 1],
                recv_sem=ag_recv.at[d - 1],
                device_id=((my + d) % N_DEV,),
                device_id_type=pl.DeviceIdType.MESH,
            )
            desc.start()
            ag.append(desc)

        row = lax.broadcasted_iota(jnp.int32, (HALF, KSPAN), 0)
        col = lax.broadcasted_iota(jnp.int32, (HALF, KSPAN), 1)
        bias_mid = jnp.where(
            (col >= row) & (col <= row + 2 * HALF), 0.0, NEG)
        bias_lo = jnp.where(jnp.abs(row - col) <= HALF, 0.0, NEG)

        def project_q(r):
            return lax.dot_general(
                x_rel[r].astype(jnp.float32), wq_vmem[...],
                (((1,), (0,)), ((), ())),
                preferred_element_type=jnp.float32,
            ) * SCALE

        def attn_half(q, r, j, hh):
            row0 = hh * HALF
            if hh == 0:
                bias = jnp.where(j == 0, bias_lo, bias_mid)
                cstart = 0
            else:
                bias = bias_mid
                cstart = jnp.where(j == 0, 0, HALF)
            ctx = []
            for h in range(H):
                qh = q[row0:row0 + HALF, h * DH:(h + 1) * DH]
                kh = k_stage[r, pl.ds(cstart, KSPAN), h, :]
                vh = v_stage[r, pl.ds(cstart, KSPAN), h, :]
                s = lax.dot_general(
                    qh, kh, (((1,), (1,)), ((), ())),
                    preferred_element_type=jnp.float32,
                )
                e = jnp.exp(s + bias)
                den = jnp.sum(e, axis=1, keepdims=True)
                ctx.append(lax.dot_general(
                    e, vh, (((1,), (0,)), ((), ())),
                    preferred_element_type=jnp.float32,
                ) / den)
            ctx = jnp.concatenate(ctx, axis=1)
            return lax.dot_general(
                ctx, wo_vmem[...], (((1,), (0,)), ((), ())),
                preferred_element_type=jnp.float32,
            )

        wqcopy.wait()
        q0 = project_q(0)
        kv[0][0].wait()
        kv[0][1].wait()
        j0 = chunk_j(0)
        h0 = attn_half(q0, 0, j0, 0)
        wocopy.wait()
        partial[0, 0:HALF] = h0.astype(jnp.bfloat16)
        partial[0, HALF:SQ] = attn_half(q0, 0, j0, 1).astype(jnp.bfloat16)

        rs = []
        for r in range(1, N_DEV):
            ag[r - 1].wait_recv()
            kv[r][0].wait()
            kv[r][1].wait()
            qr = project_q(r)
            jr = chunk_j(r)
            for hh in range(2):
                row0 = hh * HALF
                ph = attn_half(qr, r, jr, hh)
                partial[r, pl.ds(row0, HALF)] = ph.astype(jnp.bfloat16)
                sem = (r - 1) * 2 + hh
                desc = pltpu.make_async_remote_copy(
                    src_ref=partial.at[r, pl.ds(row0, HALF)],
                    dst_ref=rs_buf.at[r - 1, pl.ds(row0, HALF)],
                    send_sem=rs_send.at[sem],
                    recv_sem=rs_recv.at[sem],
                    device_id=((my + N_DEV - r) % N_DEV,),
                    device_id_type=pl.DeviceIdType.MESH,
                )
                desc.start()
                rs.append(desc)

        acc = partial[0].astype(jnp.float32)
        for k in range(N_DEV - 1):
            rs[2 * k].wait_recv()
            rs[2 * k + 1].wait_recv()
            acc = acc + rs_buf[k].astype(jnp.float32)
        out_ref[0] = acc

        for desc in ag:
            desc.wait_send()
        for desc in rs:
            desc.wait_send()

    return pl.pallas_call(
        body,
        out_shape=jax.ShapeDtypeStruct((1, SQ, D_MODEL), jnp.float32),
        in_specs=[pl.BlockSpec(memory_space=pltpu.HBM)] * 5,
        out_specs=pl.BlockSpec(memory_space=pltpu.VMEM),
        scratch_shapes=[
            pltpu.VMEM((SQ, D_MODEL), jnp.float32),
            pltpu.VMEM((D_MODEL, D_MODEL), jnp.float32),
            pltpu.VMEM((D_MODEL, D_MODEL), jnp.float32),
            pltpu.VMEM((N_DEV, KSTAGE, H, DH), jnp.float32),
            pltpu.VMEM((N_DEV, KSTAGE, H, DH), jnp.float32),
            pltpu.VMEM((N_DEV, SQ, D_MODEL), jnp.bfloat16),
            pltpu.VMEM((N_DEV, SQ, D_MODEL), jnp.bfloat16),
            pltpu.VMEM((N_DEV - 1, SQ, D_MODEL), jnp.bfloat16),
            pltpu.SemaphoreType.DMA((3,)),
            pltpu.SemaphoreType.DMA((N_DEV,)),
            pltpu.SemaphoreType.DMA((N_DEV,)),
            pltpu.SemaphoreType.DMA((N_DEV - 1,)),
            pltpu.SemaphoreType.DMA((N_DEV - 1,)),
            pltpu.SemaphoreType.DMA((2 * (N_DEV - 1),)),
            pltpu.SemaphoreType.DMA((2 * (N_DEV - 1),)),
        ],
        compiler_params=pltpu.CompilerParams(
            collective_id=0, vmem_limit_bytes=100 * 1024 * 1024),
    )(x, Wq, K_ext, V_ext, Wo)


# device time: 39249 ns/iter; 1.0012x vs baseline; 1.0012x over previous
import jax
import jax.numpy as jnp
from jax import lax
from jax.experimental import pallas as pl
from jax.experimental.pallas import tpu as pltpu

N_DEV = 4
SQ = 256
HALF = 128
D_MODEL = 1024
H = 8
DH = 128
KSTAGE = 512
KSPAN = 384
SCALE = 0.08838834764831843
NEG = -1e9


def kernel(x, Wq, K_ext, V_ext, Wo):
    def body(x_hbm, wq_hbm, k_hbm, v_hbm, wo_hbm, out_ref,
             x_vmem, wq_vmem, wo_vmem, k_stage, v_stage,
             x_rel, partial, rs_buf,
             in_sems, k_sems, v_sems, ag_send, ag_recv, rs_send, rs_recv):
        my = lax.axis_index("i")

        def chunk_j(r):
            return lax.rem(my + N_DEV - r, N_DEV)

        def key_start(j):
            return jnp.maximum(0, j * SQ - HALF)

        xcopy = pltpu.make_async_copy(x_hbm.at[0], x_vmem, in_sems.at[0])
        wqcopy = pltpu.make_async_copy(wq_hbm, wq_vmem, in_sems.at[1])
        wocopy = pltpu.make_async_copy(wo_hbm, wo_vmem, in_sems.at[2])
        xcopy.start()
        wqcopy.start()

        kv = []
        for r in range(N_DEV):
            ks = key_start(chunk_j(r))
            kc = pltpu.make_async_copy(
                k_hbm.at[0, pl.ds(ks, KSTAGE), pl.ds(my * H, H), :],
                k_stage.at[r],
                k_sems.at[r],
            )
            vc = pltpu.make_async_copy(
                v_hbm.at[0, pl.ds(ks, KSTAGE), pl.ds(my * H, H), :],
                v_stage.at[r],
                v_sems.at[r],
            )
            kv.append((kc, vc))

        kv[0][0].start()
        kv[0][1].start()
        wocopy.start()
        for r in range(1, N_DEV):
            kv[r][0].start()
            kv[r][1].start()

        barrier = pltpu.get_barrier_semaphore()
        for d in range(1, N_DEV):
            pl.semaphore_signal(
                barrier, inc=1,
                device_id=((my + d) % N_DEV,),
                device_id_type=pl.DeviceIdType.MESH,
            )

        xcopy.wait()
        x_rel[0] = x_vmem[...].astype(jnp.bfloat16)

        pl.semaphore_wait(barrier, N_DEV - 1)

        ag = []
        for d in range(1, N_DEV):
            for hh in range(2):
                desc = pltpu.make_async_remote_copy(
                    src_ref=x_rel.at[0, pl.ds(hh * HALF, HALF)],
                    dst_ref=x_rel.at[d, pl.ds(hh * HALF, HALF)],
                    send_sem=ag_send.at[(d - 1) * 2 + hh],
                    recv_sem=ag_recv.at[(d - 1) * 2 + hh],
                    device_id=((my + d) % N_DEV,),
                    device_id_type=pl.DeviceIdType.MESH,
                )
                desc.start()
                ag.append(desc)

        row = lax.broadcasted_iota(jnp.int32, (HALF, KSPAN), 0)
        col = lax.broadcasted_iota(jnp.int32, (HALF, KSPAN), 1)
        bias_mid = jnp.where(
            (col >= row) & (col <= row + 2 * HALF), 0.0, NEG)
        bias_lo = jnp.where(jnp.abs(row - col) <= HALF, 0.0, NEG)

        def project_q_half(r, hh):
            return lax.dot_general(
                x_rel[r, pl.ds(hh * HALF, HALF)].astype(jnp.float32),
                wq_vmem[...],
                (((1,), (0,)), ((), ())),
                preferred_element_type=jnp.float32,
            ) * SCALE

        def attn_half(q, r, j, hh):
            if hh == 0:
                bias = jnp.where(j == 0, bias_lo, bias_mid)
                cstart = 0
            else:
                bias = bias_mid
                cstart = jnp.where(j == 0, 0, HALF)
            ctx = []
            for h in range(H):
                qh = q[:, h * DH:(h + 1) * DH]
                kh = k_stage[r, pl.ds(cstart, KSPAN), h, :]
                vh = v_stage[r, pl.ds(cstart, KSPAN), h, :]
                s = lax.dot_general(
                    qh, kh, (((1,), (1,)), ((), ())),
                    preferred_element_type=jnp.float32,
                )
                e = jnp.exp(s + bias)
                den = jnp.sum(e, axis=1, keepdims=True)
                ctx.append(lax.dot_general(
                    e, vh, (((1,), (0,)), ((), ())),
                    preferred_element_type=jnp.float32,
                ) / den)
            ctx = jnp.concatenate(ctx, axis=1)
            return lax.dot_general(
                ctx, wo_vmem[...], (((1,), (0,)), ((), ())),
                preferred_element_type=jnp.float32,
            )

        wqcopy.wait()
        q0 = project_q_half(0, 0)
        kv[0][0].wait()
        kv[0][1].wait()
        j0 = chunk_j(0)
        h0 = attn_half(q0, 0, j0, 0)
        wocopy.wait()
        partial[0, 0:HALF] = h0.astype(jnp.bfloat16)
        partial[0, HALF:SQ] = attn_half(
            project_q_half(0, 1), 0, j0, 1).astype(jnp.bfloat16)

        rs = []
        for r in range(1, N_DEV):
            kv[r][0].wait()
            kv[r][1].wait()
            jr = chunk_j(r)
            for hh in range(2):
                row0 = hh * HALF
                ag[(r - 1) * 2 + hh].wait_recv()
                ph = attn_half(project_q_half(r, hh), r, jr, hh)
                partial[r, pl.ds(row0, HALF)] = ph.astype(jnp.bfloat16)
                sem = (r - 1) * 2 + hh
                desc = pltpu.make_async_remote_copy(
                    src_ref=partial.at[r, pl.ds(row0, HALF)],
                    dst_ref=rs_buf.at[r - 1, pl.ds(row0, HALF)],
                    send_sem=rs_send.at[sem],
                    recv_sem=rs_recv.at[sem],
                    device_id=((my + N_DEV - r) % N_DEV,),
                    device_id_type=pl.DeviceIdType.MESH,
                )
                desc.start()
                rs.append(desc)

        acc = partial[0].astype(jnp.float32)
        for k in range(N_DEV - 1):
            rs[2 * k].wait_recv()
            rs[2 * k + 1].wait_recv()
            acc = acc + rs_buf[k].astype(jnp.float32)
        out_ref[0] = acc

        for desc in ag:
            desc.wait_send()
        for desc in rs:
            desc.wait_send()

    return pl.pallas_call(
        body,
        out_shape=jax.ShapeDtypeStruct((1, SQ, D_MODEL), jnp.float32),
        in_specs=[pl.BlockSpec(memory_space=pltpu.HBM)] * 5,
        out_specs=pl.BlockSpec(memory_space=pltpu.VMEM),
        scratch_shapes=[
            pltpu.VMEM((SQ, D_MODEL), jnp.float32),
            pltpu.VMEM((D_MODEL, D_MODEL), jnp.float32),
            pltpu.VMEM((D_MODEL, D_MODEL), jnp.float32),
            pltpu.VMEM((N_DEV, KSTAGE, H, DH), jnp.float32),
            pltpu.VMEM((N_DEV, KSTAGE, H, DH), jnp.float32),
            pltpu.VMEM((N_DEV, SQ, D_MODEL), jnp.bfloat16),
            pltpu.VMEM((N_DEV, SQ, D_MODEL), jnp.bfloat16),
            pltpu.VMEM((N_DEV - 1, SQ, D_MODEL), jnp.bfloat16),
            pltpu.SemaphoreType.DMA((3,)),
            pltpu.SemaphoreType.DMA((N_DEV,)),
            pltpu.SemaphoreType.DMA((N_DEV,)),
            pltpu.SemaphoreType.DMA((2 * (N_DEV - 1),)),
            pltpu.SemaphoreType.DMA((2 * (N_DEV - 1),)),
            pltpu.SemaphoreType.DMA((2 * (N_DEV - 1),)),
            pltpu.SemaphoreType.DMA((2 * (N_DEV - 1),)),
        ],
        compiler_params=pltpu.CompilerParams(
            collective_id=0, vmem_limit_bytes=100 * 1024 * 1024),
    )(x, Wq, K_ext, V_ext, Wo)
